# baseline (speedup 1.0000x reference)
import functools

import jax
import jax.numpy as jnp
from jax import lax
from jax.experimental import pallas as pl
from jax.experimental.pallas import tpu as pltpu
from jax.experimental.pallas import tpu_sc as plsc

V = 50000
NF = 100000
NE = 3 * NF
NS = 16
NC = 2
EPT = 147 * 128
NEP = NS * EPT
NPAD = NEP - NE
NB = EPT // 16
LOG2M = 20
M = 1 << LOG2M
DUMP = M

_PAD = 128


def _i32(x):
    return jnp.int32(x - (1 << 32) if x >= (1 << 31) else x)

A_OWN = 0x8DA6B343
A_MUL = 0x85EBCA77
A_INIT = 0x9E3779B1


def _srl(x, n):
    return lax.shift_right_logical(x, jnp.full(x.shape, n, x.dtype))


def _round_body(keys3, keysf, el23, m3, amul_h,
                m3o, dsum, dcnt, avo, table,
                keyv, el2v, mv, slotv, scr1, tv, stgf, stgi, amulv):
    c = lax.axis_index("c")
    s = lax.axis_index("s")
    wid = c * NS + s
    base = s * EPT
    iota = lax.iota(jnp.int32, 16)

    pltpu.sync_copy(keys3.at[s], keyv)
    pltpu.sync_copy(el23.at[s], el2v)
    pltpu.sync_copy(m3.at[c].at[s], mv)
    pltpu.sync_copy(amul_h, amulv)
    a_mul = amulv[pl.ds(0, 16)]

    tab_base = c * (M + 16)
    def phase_a(i, _):
        k = keyv[pl.ds(i * 16, 16)]
        m = mv[pl.ds(i * 16, 16)]
        slot = _srl(k * a_mul, 32 - LOG2M)
        slotv[pl.ds(i * 16, 16)] = tab_base + jnp.where(m == 0, slot, DUMP)
        scr1[pl.ds(i * 16, 16)] = base + i * 16 + iota
        return 0

    lax.fori_loop(0, NB, phase_a, 0)

    pltpu.sync_copy(scr1, table.at[slotv])
    plsc.subcore_barrier()
    pltpu.sync_copy(table.at[slotv], tv)

    def phase_c(i, _):
        t = tv[pl.ds(i * 16, 16)]
        m = mv[pl.ds(i * 16, 16)]
        slotv[pl.ds(i * 16, 16)] = jnp.where(m == 0, t, 0)
        return 0

    lax.fori_loop(0, NB, phase_c, 0)
    pltpu.sync_copy(keysf.at[slotv], scr1)

    def phase_d(i, carry_d):
        a_s, a_c, av = carry_d
        sl = pl.ds(i * 16, 16)
        k = keyv[sl]
        m = mv[sl]
        t = tv[sl]
        kt = scr1[sl]
        e = el2v[sl]
        myidx = base + i * 16 + iota
        res = jnp.logical_and(m == 0, kt == k)
        leader = jnp.logical_and(res, t == myidx)
        a_s = a_s + jnp.where(leader, e, 0.0)
        a_c = a_c + jnp.where(leader, 1, 0)
        mnew = jnp.where(res, 1, m)
        mv[sl] = mnew
        av = av + jnp.where(mnew == 0, 1, 0)
        return a_s, a_c, av

    acc_s, acc_c, av = lax.fori_loop(
        0, NB, phase_d,
        (jnp.zeros((16,), jnp.float32), jnp.zeros((16,), jnp.int32),
         jnp.zeros((16,), jnp.int32)))

    pltpu.sync_copy(mv, m3o.at[c].at[s])
    stgf[0, pl.ds(0, 16)] = acc_s
    stgi[0, pl.ds(0, 16)] = acc_c
    stgi[1, pl.ds(0, 16)] = av
    pltpu.sync_copy(stgf.at[0], dsum.at[wid])
    pltpu.sync_copy(stgi.at[0], dcnt.at[wid])
    pltpu.sync_copy(stgi.at[1], avo.at[wid])


@functools.partial(jax.jit, static_argnames=())
def _dedup(keys3, keysf, el23, m3_0):
    mesh = plsc.VectorSubcoreMesh(core_axis_name="c", subcore_axis_name="s")
    round_call = pl.kernel(
        _round_body,
        out_type=[
            jax.ShapeDtypeStruct((NC, NS, EPT), jnp.int32),
            jax.ShapeDtypeStruct((NC * NS, 16), jnp.float32),
            jax.ShapeDtypeStruct((NC * NS, 16), jnp.int32),
            jax.ShapeDtypeStruct((NC * NS, 16), jnp.int32),
            jax.ShapeDtypeStruct((NC * (M + 16),), jnp.int32),
        ],
        mesh=mesh,
        scratch_types=[
            pltpu.VMEM((EPT,), jnp.int32),
            pltpu.VMEM((EPT,), jnp.float32),
            pltpu.VMEM((EPT,), jnp.int32),
            pltpu.VMEM((EPT,), jnp.int32),
            pltpu.VMEM((EPT,), jnp.int32),
            pltpu.VMEM((EPT,), jnp.int32),
            pltpu.VMEM((1, 16), jnp.float32),
            pltpu.VMEM((2, 16), jnp.int32),
            pltpu.VMEM((16,), jnp.int32),
        ],
    )

    def cond(carry):
        return carry[4] > 0

    def body(carry):
        m3, usum, ucnt, amul, _ = carry
        amul_vec = jnp.full((16,), amul, jnp.int32)
        m3n, ds_, dc, av, _ = round_call(keys3, keysf, el23, m3, amul_vec)
        return (m3n, usum + jnp.sum(ds_), ucnt + jnp.sum(dc),
                amul * _i32(A_MUL), jnp.sum(av))

    init = (m3_0, jnp.float32(0), jnp.int32(0), _i32(A_INIT), jnp.int32(1))
    _, usum, ucnt, _, _ = lax.while_loop(cond, body, init)
    return usum, ucnt


def _final_body(usum_ref, ucnt_ref, lvx_ref, lvy_ref, lvz_ref, nw_ref,
                vx_ref, vy_ref, vz_ref, out_ref):
    edge_sum = jnp.sum(usum_ref[...])
    edge_cnt = jnp.sum(ucnt_ref[...])
    nw = nw_ref[...]
    safe = jnp.where(nw > 0, nw, 1.0)
    inv_w = jnp.where(nw > 0, 1.0 / safe, nw)
    lx = lvx_ref[...] * inv_w - vx_ref[...]
    ly = lvy_ref[...] * inv_w - vy_ref[...]
    lz = lvz_ref[...] * inv_w - vz_ref[...]
    norms = jnp.sqrt(lx * lx + ly * ly + lz * lz)
    lap_loss = jnp.sum(norms) / V
    total = 0.1 * lap_loss + 10.0 * (edge_sum / edge_cnt)
    out_ref[...] = jnp.broadcast_to(total, (1, 1))


def _pad2d(x, n):
    return jnp.zeros((n,), x.dtype).at[: x.shape[0]].set(x).reshape(n // _PAD, _PAD)


def kernel(verts, faces):
    f0, f1, f2 = faces[:, 0], faces[:, 1], faces[:, 2]
    fv = verts[faces]
    v0, v1, v2 = fv[:, 0], fv[:, 1], fv[:, 2]
    A2 = jnp.sum((v1 - v2) ** 2, axis=1)
    B2 = jnp.sum((v0 - v2) ** 2, axis=1)
    C2 = jnp.sum((v0 - v1) ** 2, axis=1)

    def ekey(a, b):
        return jnp.minimum(a, b) * 65536 + jnp.maximum(a, b)

    keys = jnp.concatenate([ekey(f0, f1), ekey(f1, f2), ekey(f2, f0)])
    el2c = jnp.concatenate([C2, A2, B2])
    pad_keys = jnp.arange(NPAD, dtype=jnp.int32) * 65536 + 65535
    keysf = jnp.concatenate([keys, pad_keys])
    el2f = jnp.concatenate([el2c, jnp.zeros((NPAD,), jnp.float32)])
    own = (keysf * _i32(A_OWN) < 0).astype(jnp.int32)
    m3_0 = jnp.stack([(own != 0).astype(jnp.int32),
                      (own != 1).astype(jnp.int32)]).reshape(NC, NS, EPT)
    usum, ucnt = _dedup(keysf.reshape(NS, EPT), keysf, el2f.reshape(NS, EPT), m3_0)
    usum = usum.reshape(1, 1)
    ucntf = (ucnt - NPAD).astype(jnp.float32).reshape(1, 1)

    s2 = 0.5 * (A2 + B2 + C2)
    area = jnp.sqrt(jnp.clip(0.25 * (s2 * s2 - 0.5 * (A2 * A2 + B2 * B2 + C2 * C2)), 1e-12, None))
    cota = (B2 + C2 - A2) / area
    cotb = (A2 + C2 - B2) / area
    cotc = (A2 + B2 - C2) / area
    cot = jnp.stack([cota, cotb, cotc], axis=1) / 4.0
    ii = faces[:, jnp.array([1, 2, 0])].reshape(-1)
    jj = faces[:, jnp.array([2, 0, 1])].reshape(-1)
    w = cot.reshape(-1)
    Lv = jnp.zeros((V, 3), dtype=verts.dtype)
    Lv = Lv.at[ii].add(w[:, None] * verts[jj])
    Lv = Lv.at[jj].add(w[:, None] * verts[ii])
    norm_w = jnp.zeros((V,), dtype=verts.dtype)
    norm_w = norm_w.at[ii].add(w)
    norm_w = norm_w.at[jj].add(w)

    n_v = ((V + _PAD - 1) // _PAD) * _PAD
    args = [usum, ucntf,
            _pad2d(Lv[:, 0], n_v), _pad2d(Lv[:, 1], n_v), _pad2d(Lv[:, 2], n_v),
            _pad2d(norm_w, n_v),
            _pad2d(verts[:, 0], n_v), _pad2d(verts[:, 1], n_v), _pad2d(verts[:, 2], n_v)]
    out = pl.pallas_call(
        _final_body,
        out_shape=jax.ShapeDtypeStruct((1, 1), jnp.float32),
    )(*args)
    return out[0, 0]


# speedup vs baseline: 15.7310x; 15.7310x over previous
import functools

import jax
import jax.numpy as jnp
from jax import lax
from jax.experimental import pallas as pl
from jax.experimental.pallas import tpu as pltpu
from jax.experimental.pallas import tpu_sc as plsc

V = 50000
NF = 100000
NE = 3 * NF
NS = 16
NC = 2
EPT = 147 * 128
NEP = NS * EPT
NPAD = NEP - NE
NB = EPT // 16
LOG2M = 18
M = 1 << LOG2M
DUMP = M

_PAD = 128


def _i32(x):
    return jnp.int32(x - (1 << 32) if x >= (1 << 31) else x)

A_OWN = 0x8DA6B343
A_MUL = 0x85EBCA77
A_INIT = 0x9E3779B1


def _srl(x, n):
    return lax.shift_right_logical(x, jnp.full(x.shape, n, x.dtype))


def _round_body(keys3, el23, m3, amul_h,
                m3o, dsum, dcnt, avo,
                keyv, el2v, mv, slotv, scr1, stgf, stgi, amulv,
                table, keys_sh):
    c = lax.axis_index("c")
    s = lax.axis_index("s")
    wid = c * NS + s
    base = s * EPT
    iota = lax.iota(jnp.int32, 16)

    pltpu.sync_copy(keys3.at[s], keyv)
    pltpu.sync_copy(el23.at[s], el2v)
    pltpu.sync_copy(m3.at[c].at[s], mv)
    pltpu.sync_copy(amul_h, amulv)
    pltpu.sync_copy(keyv, keys_sh.at[pl.ds(base, EPT)])
    a_mul = amulv[pl.ds(0, 16)]

    def phase_a(i, _):
        k = keyv[pl.ds(i * 16, 16)]
        m = mv[pl.ds(i * 16, 16)]
        slot = _srl(k * a_mul, 32 - LOG2M)
        slotv[pl.ds(i * 16, 16)] = jnp.where(m == 0, slot, DUMP)
        scr1[pl.ds(i * 16, 16)] = base + i * 16 + iota
        return 0

    lax.fori_loop(0, NB, phase_a, 0)

    pltpu.sync_copy(scr1, table.at[slotv])
    plsc.subcore_barrier()
    pltpu.sync_copy(table.at[slotv], scr1)

    def phase_c(i, _):
        sl = pl.ds(i * 16, 16)
        t = scr1[sl]
        m = mv[sl]
        myidx = base + i * 16 + iota
        active = m == 0
        is_lead = jnp.logical_and(active, t == myidx)
        mv[sl] = jnp.where(is_lead, 2, m)
        slotv[sl] = jnp.where(jnp.logical_and(active, t != myidx), t, 0)
        return 0

    lax.fori_loop(0, NB, phase_c, 0)
    pltpu.sync_copy(keys_sh.at[slotv], scr1)

    def phase_d(i, carry_d):
        a_s, a_c, av = carry_d
        sl = pl.ds(i * 16, 16)
        k = keyv[sl]
        m = mv[sl]
        kt = scr1[sl]
        e = el2v[sl]
        lead = m == 2
        res = jnp.logical_or(lead, jnp.logical_and(m == 0, kt == k))
        a_s = a_s + jnp.where(lead, e, 0.0)
        a_c = a_c + jnp.where(lead, 1, 0)
        mnew = jnp.where(res, 1, m)
        mv[sl] = mnew
        av = av + jnp.where(mnew == 0, 1, 0)
        return a_s, a_c, av

    acc_s, acc_c, av = lax.fori_loop(
        0, NB, phase_d,
        (jnp.zeros((16,), jnp.float32), jnp.zeros((16,), jnp.int32),
         jnp.zeros((16,), jnp.int32)))

    pltpu.sync_copy(mv, m3o.at[c].at[s])
    stgf[0, pl.ds(0, 16)] = acc_s
    stgi[0, pl.ds(0, 16)] = acc_c
    stgi[1, pl.ds(0, 16)] = av
    pltpu.sync_copy(stgf.at[0], dsum.at[wid])
    pltpu.sync_copy(stgi.at[0], dcnt.at[wid])
    pltpu.sync_copy(stgi.at[1], avo.at[wid])


@functools.partial(jax.jit, static_argnames=())
def _dedup(keys3, el23, m3_0):
    mesh = plsc.VectorSubcoreMesh(core_axis_name="c", subcore_axis_name="s")
    round_call = pl.kernel(
        _round_body,
        out_type=[
            jax.ShapeDtypeStruct((NC, NS, EPT), jnp.int32),
            jax.ShapeDtypeStruct((NC * NS, 16), jnp.float32),
            jax.ShapeDtypeStruct((NC * NS, 16), jnp.int32),
            jax.ShapeDtypeStruct((NC * NS, 16), jnp.int32),
        ],
        mesh=mesh,
        scratch_types=[
            pltpu.VMEM((EPT,), jnp.int32),
            pltpu.VMEM((EPT,), jnp.float32),
            pltpu.VMEM((EPT,), jnp.int32),
            pltpu.VMEM((EPT,), jnp.int32),
            pltpu.VMEM((EPT,), jnp.int32),
            pltpu.VMEM((1, 16), jnp.float32),
            pltpu.VMEM((2, 16), jnp.int32),
            pltpu.VMEM((16,), jnp.int32),
            pltpu.VMEM_SHARED((M + 16,), jnp.int32),
            pltpu.VMEM_SHARED((NEP,), jnp.int32),
        ],
    )

    def cond(carry):
        return carry[4] > 0

    def body(carry):
        m3, usum, ucnt, amul, _ = carry
        amul_vec = jnp.full((16,), amul, jnp.int32)
        m3n, ds_, dc, av = round_call(keys3, el23, m3, amul_vec)
        return (m3n, usum + jnp.sum(ds_), ucnt + jnp.sum(dc),
                amul * _i32(A_MUL), jnp.sum(av))

    init = (m3_0, jnp.float32(0), jnp.int32(0), _i32(A_INIT), jnp.int32(1))
    _, usum, ucnt, _, _ = lax.while_loop(cond, body, init)
    return usum, ucnt


def _final_body(usum_ref, ucnt_ref, lvx_ref, lvy_ref, lvz_ref, nw_ref,
                vx_ref, vy_ref, vz_ref, out_ref):
    edge_sum = jnp.sum(usum_ref[...])
    edge_cnt = jnp.sum(ucnt_ref[...])
    nw = nw_ref[...]
    safe = jnp.where(nw > 0, nw, 1.0)
    inv_w = jnp.where(nw > 0, 1.0 / safe, nw)
    lx = lvx_ref[...] * inv_w - vx_ref[...]
    ly = lvy_ref[...] * inv_w - vy_ref[...]
    lz = lvz_ref[...] * inv_w - vz_ref[...]
    norms = jnp.sqrt(lx * lx + ly * ly + lz * lz)
    lap_loss = jnp.sum(norms) / V
    total = 0.1 * lap_loss + 10.0 * (edge_sum / edge_cnt)
    out_ref[...] = jnp.broadcast_to(total, (1, 1))


def _pad2d(x, n):
    return jnp.zeros((n,), x.dtype).at[: x.shape[0]].set(x).reshape(n // _PAD, _PAD)


def kernel(verts, faces):
    f0, f1, f2 = faces[:, 0], faces[:, 1], faces[:, 2]
    fv = verts[faces]
    v0, v1, v2 = fv[:, 0], fv[:, 1], fv[:, 2]
    A2 = jnp.sum((v1 - v2) ** 2, axis=1)
    B2 = jnp.sum((v0 - v2) ** 2, axis=1)
    C2 = jnp.sum((v0 - v1) ** 2, axis=1)

    def ekey(a, b):
        return jnp.minimum(a, b) * 65536 + jnp.maximum(a, b)

    keys = jnp.concatenate([ekey(f0, f1), ekey(f1, f2), ekey(f2, f0)])
    el2c = jnp.concatenate([C2, A2, B2])
    pad_keys = jnp.arange(NPAD, dtype=jnp.int32) * 65536 + 65535
    keysf = jnp.concatenate([keys, pad_keys])
    el2f = jnp.concatenate([el2c, jnp.zeros((NPAD,), jnp.float32)])
    own = (keysf * _i32(A_OWN) < 0).astype(jnp.int32)
    m3_0 = jnp.stack([(own != 0).astype(jnp.int32),
                      (own != 1).astype(jnp.int32)]).reshape(NC, NS, EPT)
    usum, ucnt = _dedup(keysf.reshape(NS, EPT), el2f.reshape(NS, EPT), m3_0)
    usum = usum.reshape(1, 1)
    ucntf = (ucnt - NPAD).astype(jnp.float32).reshape(1, 1)

    s2 = 0.5 * (A2 + B2 + C2)
    area = jnp.sqrt(jnp.clip(0.25 * (s2 * s2 - 0.5 * (A2 * A2 + B2 * B2 + C2 * C2)), 1e-12, None))
    cota = (B2 + C2 - A2) / area
    cotb = (A2 + C2 - B2) / area
    cotc = (A2 + B2 - C2) / area
    cot = jnp.stack([cota, cotb, cotc], axis=1) / 4.0
    ii = faces[:, jnp.array([1, 2, 0])].reshape(-1)
    jj = faces[:, jnp.array([2, 0, 1])].reshape(-1)
    w = cot.reshape(-1)
    Lv = jnp.zeros((V, 3), dtype=verts.dtype)
    Lv = Lv.at[ii].add(w[:, None] * verts[jj])
    Lv = Lv.at[jj].add(w[:, None] * verts[ii])
    norm_w = jnp.zeros((V,), dtype=verts.dtype)
    norm_w = norm_w.at[ii].add(w)
    norm_w = norm_w.at[jj].add(w)

    n_v = ((V + _PAD - 1) // _PAD) * _PAD
    args = [usum, ucntf,
            _pad2d(Lv[:, 0], n_v), _pad2d(Lv[:, 1], n_v), _pad2d(Lv[:, 2], n_v),
            _pad2d(norm_w, n_v),
            _pad2d(verts[:, 0], n_v), _pad2d(verts[:, 1], n_v), _pad2d(verts[:, 2], n_v)]
    out = pl.pallas_call(
        _final_body,
        out_shape=jax.ShapeDtypeStruct((1, 1), jnp.float32),
    )(*args)
    return out[0, 0]
